# baseline (speedup 1.0000x reference)
import functools
import numpy as np
import jax
import jax.numpy as jnp
from jax import lax
from jax.experimental import pallas as pl
from jax.experimental.pallas import tpu as pltpu
from jax.experimental.pallas import tpu_sc as plsc

_ALPHA = 4
_NC, _NS = 2, 16


def _slow_gather_sc(frames, T, n_slow, C, H, W, h_chunk):
    chunks_per_frame = H // h_chunk
    n_chunks = C * n_slow * chunks_per_frame
    NW = _NC * _NS
    assert n_chunks % NW == 0
    per_worker = n_chunks // NW
    mesh = plsc.VectorSubcoreMesh(
        core_axis_name="c", subcore_axis_name="s", num_cores=_NC, num_subcores=_NS
    )

    @functools.partial(
        pl.kernel,
        out_type=jax.ShapeDtypeStruct((C, n_slow, H, W), jnp.float32),
        mesh=mesh,
        scratch_types=[
            pltpu.VMEM((h_chunk, W), jnp.float32),
            pltpu.VMEM((h_chunk, W), jnp.float32),
            pltpu.SemaphoreType.DMA,
            pltpu.SemaphoreType.DMA,
        ],
    )
    def k(in_hbm, out_hbm, buf0, buf1, sem0, sem1):
        wid = lax.axis_index("s") * _NC + lax.axis_index("c")
        base = wid * per_worker
        bufs = (buf0, buf1)
        sems = (sem0, sem1)

        def coords(m):
            c = m // (n_slow * chunks_per_frame)
            rem = m % (n_slow * chunks_per_frame)
            j = rem // chunks_per_frame
            sub = rem % chunks_per_frame
            t = (j * (T - 1)) // (n_slow - 1)
            h0 = sub * h_chunk
            return c, j, sub, t, h0

        def in_at(m):
            c, j, sub, t, h0 = coords(m)
            return in_hbm.at[c, t, pl.ds(h0, h_chunk), :]

        def out_at(m):
            c, j, sub, t, h0 = coords(m)
            return out_hbm.at[c, j, pl.ds(h0, h_chunk), :]

        pltpu.make_async_copy(in_at(base), bufs[0], sems[0]).start()
        for kk in range(per_worker):
            m = base + kk
            b = kk % 2
            if kk + 1 < per_worker:
                nb = (kk + 1) % 2
                pltpu.make_async_copy(in_at(m + 1), bufs[nb], sems[nb]).start()
            pltpu.make_async_copy(in_at(m), bufs[b], sems[b]).wait()
            pltpu.sync_copy(bufs[b], out_at(m))

    return k(frames)


def kernel(frames):
    C, T, H, W = frames.shape
    n_slow = T // _ALPHA
    idx = np.linspace(0.0, T - 1, n_slow).astype(np.int32)
    assert all(int(v) == (j * (T - 1)) // (n_slow - 1) for j, v in enumerate(idx))

    slow = _slow_gather_sc(frames, T, n_slow, C, H, W, h_chunk=96)
    return (slow, frames)


# speedup vs baseline: 1.0468x; 1.0468x over previous
import functools
import numpy as np
import jax
import jax.numpy as jnp
from jax import lax
from jax.experimental import pallas as pl
from jax.experimental.pallas import tpu as pltpu
from jax.experimental.pallas import tpu_sc as plsc

_ALPHA = 4
_NC, _NS = 2, 16


def _slow_gather_sc(frames, T, n_slow, C, H, W, h_chunk):
    chunks_per_frame = H // h_chunk
    n_chunks = C * n_slow * chunks_per_frame
    NW = _NC * _NS
    assert n_chunks % NW == 0
    per_worker = n_chunks // NW
    mesh = plsc.VectorSubcoreMesh(
        core_axis_name="c", subcore_axis_name="s", num_cores=_NC, num_subcores=_NS
    )

    @functools.partial(
        pl.kernel,
        out_type=jax.ShapeDtypeStruct((C, n_slow, H, W), jnp.float32),
        mesh=mesh,
        scratch_types=[
            pltpu.VMEM((h_chunk, W), jnp.float32),
            pltpu.VMEM((h_chunk, W), jnp.float32),
            pltpu.SemaphoreType.DMA,
            pltpu.SemaphoreType.DMA,
        ],
    )
    def k(in_hbm, out_hbm, buf0, buf1, sem0, sem1):
        wid = lax.axis_index("s") * _NC + lax.axis_index("c")
        base = wid * per_worker
        bufs = (buf0, buf1)
        sems = (sem0, sem1)

        def coords(m):
            c = m // (n_slow * chunks_per_frame)
            rem = m % (n_slow * chunks_per_frame)
            j = rem // chunks_per_frame
            sub = rem % chunks_per_frame
            t = (j * (T - 1)) // (n_slow - 1)
            h0 = sub * h_chunk
            return c, j, sub, t, h0

        def in_at(m):
            c, j, sub, t, h0 = coords(m)
            return in_hbm.at[c, t, pl.ds(h0, h_chunk), :]

        def out_at(m):
            c, j, sub, t, h0 = coords(m)
            return out_hbm.at[c, j, pl.ds(h0, h_chunk), :]

        pltpu.make_async_copy(in_at(base), bufs[0], sems[0]).start()
        for kk in range(per_worker):
            m = base + kk
            b = kk % 2
            if kk + 1 < per_worker:
                nb = (kk + 1) % 2
                pltpu.make_async_copy(in_at(m + 1), bufs[nb], sems[nb]).start()
            pltpu.make_async_copy(in_at(m), bufs[b], sems[b]).wait()
            pltpu.sync_copy(bufs[b], out_at(m))

    return k(frames)


def _fast_copy_body(in_ref, out_ref):
    out_ref[...] = in_ref[...]


def kernel(frames):
    C, T, H, W = frames.shape
    n_slow = T // _ALPHA
    idx = np.linspace(0.0, T - 1, n_slow).astype(np.int32)
    assert all(int(v) == (j * (T - 1)) // (n_slow - 1) for j, v in enumerate(idx))

    slow = _slow_gather_sc(frames, T, n_slow, C, H, W, h_chunk=96)

    fast = pl.pallas_call(
        _fast_copy_body,
        grid=(n_slow,),
        in_specs=[pl.BlockSpec((C, _ALPHA, H, W), lambda j: (0, j, 0, 0))],
        out_specs=pl.BlockSpec((C, _ALPHA, H, W), lambda j: (0, j, 0, 0)),
        out_shape=jax.ShapeDtypeStruct((C, T, H, W), frames.dtype),
    )(frames)

    return (slow, fast)


# speedup vs baseline: 1.3400x; 1.2800x over previous
import numpy as np
import jax
import jax.numpy as jnp
from jax.experimental import pallas as pl

_ALPHA = 4
_HSPLIT = 2


def _make_body(T, n_slow):
    def body(in_ref, fast_ref, slow_ref):
        j = pl.program_id(0)
        fast_ref[...] = in_ref[...]
        off = (j * (T - 1)) // (n_slow - 1) - _ALPHA * j
        slow_ref[...] = in_ref[:, pl.ds(off, 1)]

    return body


def kernel(frames):
    C, T, H, W = frames.shape
    n_slow = T // _ALPHA
    idx = np.linspace(0.0, T - 1, n_slow).astype(np.int32)
    assert all(int(v) == (j * (T - 1)) // (n_slow - 1) for j, v in enumerate(idx))
    for j, v in enumerate(idx):
        assert _ALPHA * j <= int(v) < _ALPHA * (j + 1)
    Hb = H // _HSPLIT

    fast, slow = pl.pallas_call(
        _make_body(T, n_slow),
        grid=(n_slow, _HSPLIT),
        in_specs=[
            pl.BlockSpec((C, _ALPHA, Hb, W), lambda j, h: (0, j, h, 0)),
        ],
        out_specs=[
            pl.BlockSpec((C, _ALPHA, Hb, W), lambda j, h: (0, j, h, 0)),
            pl.BlockSpec((C, 1, Hb, W), lambda j, h: (0, j, h, 0)),
        ],
        out_shape=[
            jax.ShapeDtypeStruct((C, T, H, W), frames.dtype),
            jax.ShapeDtypeStruct((C, n_slow, H, W), frames.dtype),
        ],
    )(frames)

    return (slow, fast)
